# Initial kernel scaffold; baseline (speedup 1.0000x reference)
import jax
import jax.numpy as jnp
from jax.experimental import pallas as pl


def kernel(x, edge_index_up, target_label, W_up, b_up, W1_self, W1_nbr, b1, Wb_self, Wb_nbr, bb, Wc_self, Wc_nbr, bc):
    raise NotImplementedError("write your pallas kernel here")


# speedup vs baseline: 17.3341x; 17.3341x over previous
import functools

import jax
import jax.numpy as jnp
from jax import lax
from jax.experimental import pallas as pl
from jax.experimental.pallas import tpu as pltpu
from jax.experimental.pallas import tpu_sc as plsc

N_IN = 10000
UP = 8
N_UP = N_IN * UP
E_UP = 1280000
C_IN = 64
C_HID = 64
C_OUT = 32
HALF = 16
L_BLOCK = 3

NS = 16
K_CH = 2000


def _up_body(x_ref, w_ref, b_ref, o_ref):
    o_ref[...] = jnp.maximum(
        jnp.dot(x_ref[...], w_ref[...], preferred_element_type=jnp.float32)
        + b_ref[...], 0.0)


def _up(x, wf, bf):
    R = 1000
    return pl.pallas_call(
        _up_body,
        grid=(N_IN // R,),
        in_specs=[pl.BlockSpec((R, C_IN), lambda i: (i, 0)),
                  pl.BlockSpec((C_IN, UP * C_HID), lambda i: (0, 0)),
                  pl.BlockSpec((1, UP * C_HID), lambda i: (0, 0))],
        out_specs=pl.BlockSpec((R, UP * C_HID), lambda i: (i, 0)),
        out_shape=jax.ShapeDtypeStruct((N_IN, UP * C_HID), jnp.float32),
    )(x, wf, bf)


def _proj_body(h_ref, ws_ref, wn_ref, b_ref, s_ref, tlo_ref, thi_ref):
    hh = h_ref[...]
    s_ref[...] = (jnp.dot(hh, ws_ref[...], preferred_element_type=jnp.float32)
                  + b_ref[...])
    t = jnp.dot(hh, wn_ref[...], preferred_element_type=jnp.float32)
    tlo_ref[...] = t[:, :HALF]
    thi_ref[...] = t[:, HALF:]


def _proj(h, ws, wn, b):
    C = h.shape[1]
    R = 4000
    return pl.pallas_call(
        _proj_body,
        grid=(N_UP // R,),
        in_specs=[pl.BlockSpec((R, C), lambda i: (i, 0)),
                  pl.BlockSpec((C, C_OUT), lambda i: (0, 0)),
                  pl.BlockSpec((C, C_OUT), lambda i: (0, 0)),
                  pl.BlockSpec((1, C_OUT), lambda i: (0, 0))],
        out_specs=[pl.BlockSpec((R, C_OUT), lambda i: (i, 0)),
                   pl.BlockSpec((R, HALF), lambda i: (i, 0)),
                   pl.BlockSpec((R, HALF), lambda i: (i, 0))],
        out_shape=[jax.ShapeDtypeStruct((N_UP, C_OUT), jnp.float32),
                   jax.ShapeDtypeStruct((N_UP, HALF), jnp.float32),
                   jax.ShapeDtypeStruct((N_UP, HALF), jnp.float32)],
    )(h, ws, wn, b)


def _combine_body(s_ref, ylo_ref, yhi_ref, o_ref):
    o_ref[:, :HALF] = jnp.maximum(s_ref[:, :HALF] + ylo_ref[...], 0.0)
    o_ref[:, HALF:] = jnp.maximum(s_ref[:, HALF:] + yhi_ref[...], 0.0)


def _combine(s, ylo, yhi):
    R = 4000
    return pl.pallas_call(
        _combine_body,
        grid=(N_UP // R,),
        in_specs=[pl.BlockSpec((R, C_OUT), lambda i: (i, 0)),
                  pl.BlockSpec((R, HALF), lambda i: (i, 0)),
                  pl.BlockSpec((R, HALF), lambda i: (i, 0))],
        out_specs=pl.BlockSpec((R, C_OUT), lambda i: (i, 0)),
        out_shape=jax.ShapeDtypeStruct((N_UP, C_OUT), jnp.float32),
    )(s, ylo, yhi)


def _combine_res_body(hr_ref, s_ref, ylo_ref, yhi_ref, o_ref):
    o_ref[:, :HALF] = jnp.maximum(
        hr_ref[:, :HALF] + s_ref[:, :HALF] + ylo_ref[...], 0.0)
    o_ref[:, HALF:] = jnp.maximum(
        hr_ref[:, HALF:] + s_ref[:, HALF:] + yhi_ref[...], 0.0)


def _combine_res(hr, s, ylo, yhi):
    R = 4000
    return pl.pallas_call(
        _combine_res_body,
        grid=(N_UP // R,),
        in_specs=[pl.BlockSpec((R, C_OUT), lambda i: (i, 0)),
                  pl.BlockSpec((R, C_OUT), lambda i: (i, 0)),
                  pl.BlockSpec((R, HALF), lambda i: (i, 0)),
                  pl.BlockSpec((R, HALF), lambda i: (i, 0))],
        out_specs=pl.BlockSpec((R, C_OUT), lambda i: (i, 0)),
        out_shape=jax.ShapeDtypeStruct((N_UP, C_OUT), jnp.float32),
    )(hr, s, ylo, yhi)


_NGRP = N_UP // C_OUT
_GW = C_OUT * HALF


def _cls_body(h_ref, wds_ref, wf_ref, bc_ref, s_ref, t_ref):
    hh = h_ref[...]
    s_ref[...] = (jnp.dot(hh, wds_ref[...], preferred_element_type=jnp.float32)
                  + bc_ref[0, 0])
    t_ref[...] = jnp.dot(hh, wf_ref[...], preferred_element_type=jnp.float32)


def _cls_proj(h32, wds, wfold, bc):
    return pl.pallas_call(
        _cls_body,
        in_specs=[pl.BlockSpec((_NGRP, C_OUT * C_OUT), lambda: (0, 0)),
                  pl.BlockSpec((C_OUT * C_OUT, C_OUT), lambda: (0, 0)),
                  pl.BlockSpec((C_OUT * C_OUT, _GW), lambda: (0, 0)),
                  pl.BlockSpec((1, 1), lambda: (0, 0))],
        out_specs=[pl.BlockSpec((_NGRP, C_OUT), lambda: (0, 0)),
                   pl.BlockSpec((_NGRP, _GW), lambda: (0, 0))],
        out_shape=[jax.ShapeDtypeStruct((_NGRP, C_OUT), jnp.float32),
                   jax.ShapeDtypeStruct((_NGRP, _GW), jnp.float32)],
    )(h32, wds, wfold, bc)


def _reduce_body(s_ref, y0_ref, y1_ref, p2_ref, lg_ref, mx_ref, top_ref):
    p2 = p2_ref[...]
    lg = (s_ref[...]
          + jnp.dot(y0_ref[...], p2, preferred_element_type=jnp.float32)
          + jnp.dot(y1_ref[...], p2, preferred_element_type=jnp.float32))
    lg_ref[...] = lg
    mx = jnp.max(lg)
    mx_ref[...] = jnp.reshape(mx, (1, 1))
    flat = (lax.broadcasted_iota(jnp.int32, (_NGRP, C_OUT), 0) * C_OUT
            + lax.broadcasted_iota(jnp.int32, (_NGRP, C_OUT), 1))
    top_ref[...] = jnp.reshape(
        jnp.min(jnp.where(lg == mx, flat, jnp.int32(2**30))), (1, 1))


def _reduce(sC, y0r, y1r, p2):
    return pl.pallas_call(
        _reduce_body,
        in_specs=[pl.BlockSpec((_NGRP, C_OUT), lambda: (0, 0)),
                  pl.BlockSpec((_NGRP, _GW), lambda: (0, 0)),
                  pl.BlockSpec((_NGRP, _GW), lambda: (0, 0)),
                  pl.BlockSpec((_GW, C_OUT), lambda: (0, 0))],
        out_specs=[pl.BlockSpec((_NGRP, C_OUT), lambda: (0, 0)),
                   pl.BlockSpec((1, 1), lambda: (0, 0)),
                   pl.BlockSpec((1, 1), lambda: (0, 0))],
        out_shape=[jax.ShapeDtypeStruct((_NGRP, C_OUT), jnp.float32),
                   jax.ShapeDtypeStruct((1, 1), jnp.float32),
                   jax.ShapeDtypeStruct((1, 1), jnp.int32)],
    )(sC, y0r, y1r, p2)


def _mask_body(h_ref, lg_ref, tgt_ref, mx_ref, top_ref, o_ref, k_ref):
    R = 4000
    pid = pl.program_id(0)
    lg = lg_ref[...]
    iota = lax.broadcasted_iota(jnp.int32, (R, 1), 0) + pid * R
    keep = ((lg > 0.0) | (tgt_ref[...] != 0)
            | ((iota == top_ref[0, 0]) & (mx_ref[0, 0] < 0.0)))
    o_ref[...] = h_ref[...] * keep.astype(jnp.float32)
    k_ref[...] = keep.astype(jnp.int32)


def _mask(h, lgN, tgtN, mx, top):
    R = 4000
    return pl.pallas_call(
        _mask_body,
        grid=(N_UP // R,),
        in_specs=[pl.BlockSpec((R, C_OUT), lambda i: (i, 0)),
                  pl.BlockSpec((R, 1), lambda i: (i, 0)),
                  pl.BlockSpec((R, 1), lambda i: (i, 0)),
                  pl.BlockSpec((1, 1), lambda i: (0, 0)),
                  pl.BlockSpec((1, 1), lambda i: (0, 0))],
        out_specs=[pl.BlockSpec((R, C_OUT), lambda i: (i, 0)),
                   pl.BlockSpec((R, 1), lambda i: (i, 0))],
        out_shape=[jax.ShapeDtypeStruct((N_UP, C_OUT), jnp.float32),
                   jax.ShapeDtypeStruct((N_UP, 1), jnp.int32)],
    )(h, lgN, tgtN, mx, top)


_MESH = plsc.VectorSubcoreMesh(core_axis_name="c", subcore_axis_name="s",
                               num_cores=2, num_subcores=NS)

_SC_SCRATCH = [
    pltpu.VMEM((K_CH,), jnp.int32),
    pltpu.VMEM((K_CH,), jnp.int32),
    pltpu.VMEM((K_CH, HALF), jnp.float32),
    pltpu.VMEM_SHARED((N_UP, HALF), jnp.float32),
    pltpu.SemaphoreType.DMA,
]


def _edge_loop(table_h, src_h, dst_h, acc, srcv, dstv, rows, sem, base0, nch):
    def chunk(g, carry):
        base = base0 + g * K_CH
        pltpu.sync_copy(src_h.at[pl.ds(base, K_CH)], srcv)
        pltpu.sync_copy(dst_h.at[pl.ds(base, K_CH)], dstv)
        pltpu.async_copy(table_h.at[srcv], rows, sem).wait()
        pltpu.sync_copy(rows, acc.at[dstv], add=True)
        return carry
    lax.fori_loop(0, nch, chunk, 0)


@functools.partial(
    pl.kernel,
    out_type=[jax.ShapeDtypeStruct((N_UP, HALF), jnp.float32)] * 2,
    mesh=_MESH,
    scratch_types=_SC_SCRATCH,
    compiler_params=pltpu.CompilerParams(use_tc_tiling_on_sc=False),
)
def _sc_wide(tlo_h, thi_h, src_h, dst_h, zer_h, ylo_h, yhi_h,
             srcv, dstv, rows, acc, sem):
    cid = lax.axis_index("c")
    sid = lax.axis_index("s")
    rpt = N_UP // NS
    row0 = sid * rpt
    pltpu.sync_copy(zer_h.at[pl.ds(row0, rpt)], acc.at[pl.ds(row0, rpt)])
    plsc.subcore_barrier()
    ept = E_UP // NS
    nch = ept // K_CH

    @pl.when(cid == 0)
    def _():
        _edge_loop(tlo_h, src_h, dst_h, acc, srcv, dstv, rows, sem,
                   sid * ept, nch)

    @pl.when(cid == 1)
    def _():
        _edge_loop(thi_h, src_h, dst_h, acc, srcv, dstv, rows, sem,
                   sid * ept, nch)

    plsc.subcore_barrier()

    @pl.when(cid == 0)
    def _():
        pltpu.sync_copy(acc.at[pl.ds(row0, rpt)], ylo_h.at[pl.ds(row0, rpt)])

    @pl.when(cid == 1)
    def _():
        pltpu.sync_copy(acc.at[pl.ds(row0, rpt)], yhi_h.at[pl.ds(row0, rpt)])


@functools.partial(
    pl.kernel,
    out_type=[jax.ShapeDtypeStruct((N_UP, HALF), jnp.float32)] * 2,
    mesh=_MESH,
    scratch_types=_SC_SCRATCH,
    compiler_params=pltpu.CompilerParams(use_tc_tiling_on_sc=False),
)
def _sc_cls(t_h, src_h, dst_h, zer_h, y0_h, y1_h, srcv, dstv, rows, acc, sem):
    cid = lax.axis_index("c")
    sid = lax.axis_index("s")
    rpt = N_UP // NS
    row0 = sid * rpt
    pltpu.sync_copy(zer_h.at[pl.ds(row0, rpt)], acc.at[pl.ds(row0, rpt)])
    plsc.subcore_barrier()
    ept = E_UP // (2 * NS)
    nch = ept // K_CH
    _edge_loop(t_h, src_h, dst_h, acc, srcv, dstv, rows, sem,
               cid * (E_UP // 2) + sid * ept, nch)
    plsc.subcore_barrier()

    @pl.when(cid == 0)
    def _():
        pltpu.sync_copy(acc.at[pl.ds(row0, rpt)], y0_h.at[pl.ds(row0, rpt)])

    @pl.when(cid == 1)
    def _():
        pltpu.sync_copy(acc.at[pl.ds(row0, rpt)], y1_h.at[pl.ds(row0, rpt)])


def kernel(x, edge_index_up, target_label, W_up, b_up, W1_self, W1_nbr, b1,
           Wb_self, Wb_nbr, bb, Wc_self, Wc_nbr, bc):
    f32 = jnp.float32
    src = edge_index_up[0]
    dst = edge_index_up[1]
    zer = jnp.zeros((N_UP, HALF), f32)

    wf_up = jnp.transpose(W_up, (1, 0, 2)).reshape(C_IN, UP * C_HID)
    bf_up = jnp.tile(b_up, UP).reshape(1, UP * C_HID)
    h = _up(x, wf_up, bf_up).reshape(N_UP, C_HID)

    s, tlo, thi = _proj(h, W1_self, W1_nbr, b1.reshape(1, C_OUT))
    ylo, yhi = _sc_wide(tlo, thi, src, dst, zer)
    h = _combine(s, ylo, yhi)

    for l in range(L_BLOCK):
        s, tlo, thi = _proj(h, Wb_self[l, 0], Wb_nbr[l, 0],
                            bb[l, 0].reshape(1, C_OUT))
        ylo, yhi = _sc_wide(tlo, thi, src, dst, zer)
        r = _combine(s, ylo, yhi)
        s, tlo, thi = _proj(r, Wb_self[l, 1], Wb_nbr[l, 1],
                            bb[l, 1].reshape(1, C_OUT))
        ylo, yhi = _sc_wide(tlo, thi, src, dst, zer)
        h = _combine_res(h, s, ylo, yhi)

    eye = jnp.eye(C_OUT, dtype=f32)
    bds = jnp.einsum('jk,c->jck', eye, Wc_self[:, 0]).reshape(
        C_OUT * C_OUT, C_OUT)
    bdn = jnp.einsum('jk,c->jck', eye, Wc_nbr[:, 0]).reshape(
        C_OUT * C_OUT, C_OUT)
    P = jnp.zeros((C_OUT, _GW), f32).at[
        jnp.arange(C_OUT), jnp.arange(C_OUT) * HALF].set(1.0)
    wfold = bdn @ P
    h32 = h.reshape(_NGRP, C_OUT * C_OUT)
    sC, tpadw = _cls_proj(h32, bds, wfold, bc.reshape(1, 1))
    tpad = tpadw.reshape(N_UP, HALF)
    y0, y1 = _sc_cls(tpad, src, dst, zer)

    lg32, mx, top = _reduce(sC, y0.reshape(_NGRP, _GW),
                            y1.reshape(_NGRP, _GW), P.T)
    out_cls = lg32.reshape(N_UP, 1)
    tgtN = target_label.astype(jnp.int32).reshape(N_UP, 1)
    out_pruned, keep_i = _mask(h, out_cls, tgtN, mx, top)
    keep = keep_i.reshape(N_UP) != 0
    return out_pruned, out_cls, target_label, keep


# speedup vs baseline: 28.0459x; 1.6180x over previous
import functools

import jax
import jax.numpy as jnp
from jax import lax
from jax.experimental import pallas as pl
from jax.experimental.pallas import tpu as pltpu
from jax.experimental.pallas import tpu_sc as plsc

N_IN = 10000
UP = 8
N_UP = N_IN * UP
E_UP = 1280000
C_IN = 64
C_HID = 64
C_OUT = 32
HALF = 16
L_BLOCK = 3

G_HID = UP * C_HID
G_OUT = UP * C_OUT
G_HALF = UP * HALF

NS = 16
K_CH = 2000

_R = 2000
_NG = N_IN // _R


def _up_body(x_ref, w_ref, b_ref, o_ref):
    o_ref[...] = jnp.maximum(
        jnp.dot(x_ref[...], w_ref[...], preferred_element_type=jnp.float32)
        + b_ref[...], 0.0)


def _up(x, wf, bf):
    R = 1000
    return pl.pallas_call(
        _up_body,
        grid=(N_IN // R,),
        in_specs=[pl.BlockSpec((R, C_IN), lambda i: (i, 0)),
                  pl.BlockSpec((C_IN, G_HID), lambda i: (0, 0)),
                  pl.BlockSpec((1, G_HID), lambda i: (0, 0))],
        out_specs=pl.BlockSpec((R, G_HID), lambda i: (i, 0)),
        out_shape=jax.ShapeDtypeStruct((N_IN, G_HID), jnp.float32),
    )(x, wf, bf)


def _proj0_body(h_ref, ws_ref, wlo_ref, whi_ref, b_ref,
                s_ref, tlo_ref, thi_ref):
    hh = h_ref[...]
    s_ref[...] = (jnp.dot(hh, ws_ref[...], preferred_element_type=jnp.float32)
                  + b_ref[...])
    tlo_ref[...] = jnp.dot(hh, wlo_ref[...], preferred_element_type=jnp.float32)
    thi_ref[...] = jnp.dot(hh, whi_ref[...], preferred_element_type=jnp.float32)


def _proj0(h, ws, wlo, whi, b):
    C = h.shape[1]
    return pl.pallas_call(
        _proj0_body,
        grid=(_NG,),
        in_specs=[pl.BlockSpec((_R, C), lambda i: (i, 0)),
                  pl.BlockSpec((C, G_OUT), lambda i: (0, 0)),
                  pl.BlockSpec((C, G_HALF), lambda i: (0, 0)),
                  pl.BlockSpec((C, G_HALF), lambda i: (0, 0)),
                  pl.BlockSpec((1, G_OUT), lambda i: (0, 0))],
        out_specs=[pl.BlockSpec((_R, G_OUT), lambda i: (i, 0)),
                   pl.BlockSpec((_R, G_HALF), lambda i: (i, 0)),
                   pl.BlockSpec((_R, G_HALF), lambda i: (i, 0))],
        out_shape=[jax.ShapeDtypeStruct((N_IN, G_OUT), jnp.float32),
                   jax.ShapeDtypeStruct((N_IN, G_HALF), jnp.float32),
                   jax.ShapeDtypeStruct((N_IN, G_HALF), jnp.float32)],
    )(h, ws, wlo, whi, b)


def _cproj_body(s_ref, ylo_ref, yhi_ref, plo_ref, phi_ref,
                ws_ref, wlo_ref, whi_ref, b_ref,
                h_ref, s2_ref, tlo_ref, thi_ref):
    y = (jnp.dot(ylo_ref[...], plo_ref[...], preferred_element_type=jnp.float32)
         + jnp.dot(yhi_ref[...], phi_ref[...], preferred_element_type=jnp.float32))
    h = jnp.maximum(s_ref[...] + y, 0.0)
    h_ref[...] = h
    s2_ref[...] = (jnp.dot(h, ws_ref[...], preferred_element_type=jnp.float32)
                   + b_ref[...])
    tlo_ref[...] = jnp.dot(h, wlo_ref[...], preferred_element_type=jnp.float32)
    thi_ref[...] = jnp.dot(h, whi_ref[...], preferred_element_type=jnp.float32)


def _cproj_res_body(res_ref, s_ref, ylo_ref, yhi_ref, plo_ref, phi_ref,
                    ws_ref, wlo_ref, whi_ref, b_ref,
                    h_ref, s2_ref, tlo_ref, thi_ref):
    y = (jnp.dot(ylo_ref[...], plo_ref[...], preferred_element_type=jnp.float32)
         + jnp.dot(yhi_ref[...], phi_ref[...], preferred_element_type=jnp.float32))
    h = jnp.maximum(res_ref[...] + s_ref[...] + y, 0.0)
    h_ref[...] = h
    s2_ref[...] = (jnp.dot(h, ws_ref[...], preferred_element_type=jnp.float32)
                   + b_ref[...])
    tlo_ref[...] = jnp.dot(h, wlo_ref[...], preferred_element_type=jnp.float32)
    thi_ref[...] = jnp.dot(h, whi_ref[...], preferred_element_type=jnp.float32)


def _row_spec(w):
    return pl.BlockSpec((_R, w), lambda i: (i, 0))


def _full_spec(r, w):
    return pl.BlockSpec((r, w), lambda i: (0, 0))


_CPROJ_OUT = [jax.ShapeDtypeStruct((N_IN, G_OUT), jnp.float32),
              jax.ShapeDtypeStruct((N_IN, G_OUT), jnp.float32),
              jax.ShapeDtypeStruct((N_IN, G_HALF), jnp.float32),
              jax.ShapeDtypeStruct((N_IN, G_HALF), jnp.float32)]

_CPROJ_OUT_SPECS = [pl.BlockSpec((_R, G_OUT), lambda i: (i, 0)),
                    pl.BlockSpec((_R, G_OUT), lambda i: (i, 0)),
                    pl.BlockSpec((_R, G_HALF), lambda i: (i, 0)),
                    pl.BlockSpec((_R, G_HALF), lambda i: (i, 0))]


def _cproj(s, ylo, yhi, plo, phi, ws, wlo, whi, b):
    return pl.pallas_call(
        _cproj_body,
        grid=(_NG,),
        in_specs=[_row_spec(G_OUT), _row_spec(G_HALF), _row_spec(G_HALF),
                  _full_spec(G_HALF, G_OUT), _full_spec(G_HALF, G_OUT),
                  _full_spec(G_OUT, G_OUT), _full_spec(G_OUT, G_HALF),
                  _full_spec(G_OUT, G_HALF), _full_spec(1, G_OUT)],
        out_specs=_CPROJ_OUT_SPECS,
        out_shape=_CPROJ_OUT,
    )(s, ylo, yhi, plo, phi, ws, wlo, whi, b)


def _cproj_res(res, s, ylo, yhi, plo, phi, ws, wlo, whi, b):
    return pl.pallas_call(
        _cproj_res_body,
        grid=(_NG,),
        in_specs=[_row_spec(G_OUT),
                  _row_spec(G_OUT), _row_spec(G_HALF), _row_spec(G_HALF),
                  _full_spec(G_HALF, G_OUT), _full_spec(G_HALF, G_OUT),
                  _full_spec(G_OUT, G_OUT), _full_spec(G_OUT, G_HALF),
                  _full_spec(G_OUT, G_HALF), _full_spec(1, G_OUT)],
        out_specs=_CPROJ_OUT_SPECS,
        out_shape=_CPROJ_OUT,
    )(res, s, ylo, yhi, plo, phi, ws, wlo, whi, b)


def _ccls_body(res_ref, s_ref, ylo_ref, yhi_ref, plo_ref, phi_ref,
               wcs_ref, wcn_ref, bc_ref,
               h_ref, sc_ref, tpad_ref):
    y = (jnp.dot(ylo_ref[...], plo_ref[...], preferred_element_type=jnp.float32)
         + jnp.dot(yhi_ref[...], phi_ref[...], preferred_element_type=jnp.float32))
    h = jnp.maximum(res_ref[...] + s_ref[...] + y, 0.0)
    h_ref[...] = h
    sc_ref[...] = (jnp.dot(h, wcs_ref[...], preferred_element_type=jnp.float32)
                   + bc_ref[...])
    tpad_ref[...] = jnp.dot(h, wcn_ref[...], preferred_element_type=jnp.float32)


def _ccls(res, s, ylo, yhi, plo, phi, wcs, wcn, bc):
    return pl.pallas_call(
        _ccls_body,
        grid=(_NG,),
        in_specs=[_row_spec(G_OUT),
                  _row_spec(G_OUT), _row_spec(G_HALF), _row_spec(G_HALF),
                  _full_spec(G_HALF, G_OUT), _full_spec(G_HALF, G_OUT),
                  _full_spec(G_OUT, G_HALF), _full_spec(G_OUT, G_HALF),
                  _full_spec(1, G_HALF)],
        out_specs=[pl.BlockSpec((_R, G_OUT), lambda i: (i, 0)),
                   pl.BlockSpec((_R, G_HALF), lambda i: (i, 0)),
                   pl.BlockSpec((_R, G_HALF), lambda i: (i, 0))],
        out_shape=[jax.ShapeDtypeStruct((N_IN, G_OUT), jnp.float32),
                   jax.ShapeDtypeStruct((N_IN, G_HALF), jnp.float32),
                   jax.ShapeDtypeStruct((N_IN, G_HALF), jnp.float32)],
    )(res, s, ylo, yhi, plo, phi, wcs, wcn, bc)


def _finreduce_body(sc_ref, y0_ref, y1_ref, lg_ref, mx_ref, top_ref):
    lg = sc_ref[...] + y0_ref[...] + y1_ref[...]
    lg_ref[...] = lg
    col = lax.broadcasted_iota(jnp.int32, (N_IN, G_HALF), 1)
    node = (lax.broadcasted_iota(jnp.int32, (N_IN, G_HALF), 0) * UP
            + col // HALF)
    valid = (col % HALF) == 0
    neg = jnp.float32(-3.0e38)
    mx = jnp.max(jnp.where(valid, lg, neg))
    mx_ref[...] = jnp.reshape(mx, (1, 1))
    top_ref[...] = jnp.reshape(
        jnp.min(jnp.where(valid & (lg == mx), node, jnp.int32(2**30))), (1, 1))


def _finreduce(sc, y0, y1):
    return pl.pallas_call(
        _finreduce_body,
        in_specs=[pl.BlockSpec((N_IN, G_HALF), lambda: (0, 0))] * 3,
        out_specs=[pl.BlockSpec((N_IN, G_HALF), lambda: (0, 0)),
                   pl.BlockSpec((1, 1), lambda: (0, 0)),
                   pl.BlockSpec((1, 1), lambda: (0, 0))],
        out_shape=[jax.ShapeDtypeStruct((N_IN, G_HALF), jnp.float32),
                   jax.ShapeDtypeStruct((1, 1), jnp.float32),
                   jax.ShapeDtypeStruct((1, 1), jnp.int32)],
    )(sc, y0, y1)


def _mask_body(h_ref, lg_ref, tgt_ref, mx_ref, top_ref, o_ref, k_ref):
    R = 4000
    pid = pl.program_id(0)
    lg = lg_ref[...]
    iota = lax.broadcasted_iota(jnp.int32, (R, 1), 0) + pid * R
    keep = ((lg > 0.0) | (tgt_ref[...] != 0)
            | ((iota == top_ref[0, 0]) & (mx_ref[0, 0] < 0.0)))
    o_ref[...] = h_ref[...] * keep.astype(jnp.float32)
    k_ref[...] = keep.astype(jnp.int32)


def _mask(h, lgN, tgtN, mx, top):
    R = 4000
    return pl.pallas_call(
        _mask_body,
        grid=(N_UP // R,),
        in_specs=[pl.BlockSpec((R, C_OUT), lambda i: (i, 0)),
                  pl.BlockSpec((R, 1), lambda i: (i, 0)),
                  pl.BlockSpec((R, 1), lambda i: (i, 0)),
                  pl.BlockSpec((1, 1), lambda i: (0, 0)),
                  pl.BlockSpec((1, 1), lambda i: (0, 0))],
        out_specs=[pl.BlockSpec((R, C_OUT), lambda i: (i, 0)),
                   pl.BlockSpec((R, 1), lambda i: (i, 0))],
        out_shape=[jax.ShapeDtypeStruct((N_UP, C_OUT), jnp.float32),
                   jax.ShapeDtypeStruct((N_UP, 1), jnp.int32)],
    )(h, lgN, tgtN, mx, top)


_MESH = plsc.VectorSubcoreMesh(core_axis_name="c", subcore_axis_name="s",
                               num_cores=2, num_subcores=NS)

_SC_SCRATCH = [
    pltpu.VMEM((K_CH,), jnp.int32),
    pltpu.VMEM((K_CH,), jnp.int32),
    pltpu.VMEM((K_CH, HALF), jnp.float32),
    pltpu.VMEM_SHARED((N_UP, HALF), jnp.float32),
    pltpu.SemaphoreType.DMA,
]


def _edge_loop(table_h, src_h, dst_h, acc, srcv, dstv, rows, sem, base0, nch):
    def chunk(g, carry):
        base = base0 + g * K_CH
        pltpu.sync_copy(src_h.at[pl.ds(base, K_CH)], srcv)
        pltpu.sync_copy(dst_h.at[pl.ds(base, K_CH)], dstv)
        pltpu.async_copy(table_h.at[srcv], rows, sem).wait()
        pltpu.sync_copy(rows, acc.at[dstv], add=True)
        return carry
    lax.fori_loop(0, nch, chunk, 0)


@functools.partial(
    pl.kernel,
    out_type=[jax.ShapeDtypeStruct((N_UP, HALF), jnp.float32)] * 2,
    mesh=_MESH,
    scratch_types=_SC_SCRATCH,
    compiler_params=pltpu.CompilerParams(use_tc_tiling_on_sc=False),
)
def _sc_wide(tlo_h, thi_h, src_h, dst_h, zer_h, ylo_h, yhi_h,
             srcv, dstv, rows, acc, sem):
    cid = lax.axis_index("c")
    sid = lax.axis_index("s")
    rpt = N_UP // NS
    row0 = sid * rpt
    pltpu.sync_copy(zer_h.at[pl.ds(row0, rpt)], acc.at[pl.ds(row0, rpt)])
    plsc.subcore_barrier()
    ept = E_UP // NS
    nch = ept // K_CH

    @pl.when(cid == 0)
    def _():
        _edge_loop(tlo_h, src_h, dst_h, acc, srcv, dstv, rows, sem,
                   sid * ept, nch)

    @pl.when(cid == 1)
    def _():
        _edge_loop(thi_h, src_h, dst_h, acc, srcv, dstv, rows, sem,
                   sid * ept, nch)

    plsc.subcore_barrier()

    @pl.when(cid == 0)
    def _():
        pltpu.sync_copy(acc.at[pl.ds(row0, rpt)], ylo_h.at[pl.ds(row0, rpt)])

    @pl.when(cid == 1)
    def _():
        pltpu.sync_copy(acc.at[pl.ds(row0, rpt)], yhi_h.at[pl.ds(row0, rpt)])


@functools.partial(
    pl.kernel,
    out_type=[jax.ShapeDtypeStruct((N_UP, HALF), jnp.float32)] * 2,
    mesh=_MESH,
    scratch_types=_SC_SCRATCH,
    compiler_params=pltpu.CompilerParams(use_tc_tiling_on_sc=False),
)
def _sc_cls(t_h, src_h, dst_h, zer_h, y0_h, y1_h, srcv, dstv, rows, acc, sem):
    cid = lax.axis_index("c")
    sid = lax.axis_index("s")
    rpt = N_UP // NS
    row0 = sid * rpt
    pltpu.sync_copy(zer_h.at[pl.ds(row0, rpt)], acc.at[pl.ds(row0, rpt)])
    plsc.subcore_barrier()
    ept = E_UP // (2 * NS)
    nch = ept // K_CH
    _edge_loop(t_h, src_h, dst_h, acc, srcv, dstv, rows, sem,
               cid * (E_UP // 2) + sid * ept, nch)
    plsc.subcore_barrier()

    @pl.when(cid == 0)
    def _():
        pltpu.sync_copy(acc.at[pl.ds(row0, rpt)], y0_h.at[pl.ds(row0, rpt)])

    @pl.when(cid == 1)
    def _():
        pltpu.sync_copy(acc.at[pl.ds(row0, rpt)], y1_h.at[pl.ds(row0, rpt)])


def _bd8(w):
    c, d = w.shape
    return jnp.einsum('jk,cd->jckd', jnp.eye(UP, dtype=w.dtype), w).reshape(
        UP * c, UP * d)


def _grouped(a):
    return a.reshape(N_IN, G_HALF)


def _flat16(a):
    return a.reshape(N_UP, HALF)


def kernel(x, edge_index_up, target_label, W_up, b_up, W1_self, W1_nbr, b1,
           Wb_self, Wb_nbr, bb, Wc_self, Wc_nbr, bc):
    f32 = jnp.float32
    src = edge_index_up[0]
    dst = edge_index_up[1]
    zer = jnp.zeros((N_UP, HALF), f32)

    plo = _bd8(jnp.concatenate(
        [jnp.eye(HALF, dtype=f32), jnp.zeros((HALF, HALF), f32)], axis=1))
    phi = _bd8(jnp.concatenate(
        [jnp.zeros((HALF, HALF), f32), jnp.eye(HALF, dtype=f32)], axis=1))

    wf_up = jnp.transpose(W_up, (1, 0, 2)).reshape(C_IN, G_HID)
    bf_up = jnp.tile(b_up, UP).reshape(1, G_HID)
    h = _up(x, wf_up, bf_up)

    s, tlo, thi = _proj0(h, _bd8(W1_self), _bd8(W1_nbr[:, :HALF]),
                         _bd8(W1_nbr[:, HALF:]),
                         jnp.tile(b1, UP).reshape(1, G_OUT))
    ylo, yhi = _sc_wide(_flat16(tlo), _flat16(thi), src, dst, zer)

    hres, s, tlo, thi = _cproj(
        s, _grouped(ylo), _grouped(yhi), plo, phi,
        _bd8(Wb_self[0, 0]), _bd8(Wb_nbr[0, 0][:, :HALF]),
        _bd8(Wb_nbr[0, 0][:, HALF:]), jnp.tile(bb[0, 0], UP).reshape(1, G_OUT))
    ylo, yhi = _sc_wide(_flat16(tlo), _flat16(thi), src, dst, zer)

    for l in range(L_BLOCK):
        _, s, tlo, thi = _cproj(
            s, _grouped(ylo), _grouped(yhi), plo, phi,
            _bd8(Wb_self[l, 1]), _bd8(Wb_nbr[l, 1][:, :HALF]),
            _bd8(Wb_nbr[l, 1][:, HALF:]),
            jnp.tile(bb[l, 1], UP).reshape(1, G_OUT))
        ylo, yhi = _sc_wide(_flat16(tlo), _flat16(thi), src, dst, zer)
        if l < L_BLOCK - 1:
            hres, s, tlo, thi = _cproj_res(
                hres, s, _grouped(ylo), _grouped(yhi), plo, phi,
                _bd8(Wb_self[l + 1, 0]), _bd8(Wb_nbr[l + 1, 0][:, :HALF]),
                _bd8(Wb_nbr[l + 1, 0][:, HALF:]),
                jnp.tile(bb[l + 1, 0], UP).reshape(1, G_OUT))
            ylo, yhi = _sc_wide(_flat16(tlo), _flat16(thi), src, dst, zer)

    e0 = jnp.zeros((1, HALF), f32).at[0, 0].set(1.0)
    wcs = _bd8(Wc_self @ e0)
    wcn = _bd8(Wc_nbr @ e0)
    bc_g = jnp.tile(bc[0] * e0, (1, UP))
    hfin, sc_g, tpad = _ccls(hres, s, _grouped(ylo), _grouped(yhi), plo, phi,
                             wcs, wcn, bc_g)
    y0, y1 = _sc_cls(_flat16(tpad), src, dst, zer)

    lg_g, mx, top = _finreduce(sc_g, _grouped(y0), _grouped(y1))

    out_cls = lg_g.reshape(N_UP, HALF)[:, :1]
    hN = hfin.reshape(N_UP, C_OUT)
    tgtN = target_label.astype(jnp.int32).reshape(N_UP, 1)
    out_pruned, keep_i = _mask(hN, out_cls, tgtN, mx, top)
    keep = keep_i.reshape(N_UP) != 0
    return out_pruned, out_cls, target_label, keep


# speedup vs baseline: 36.1285x; 1.2882x over previous
import functools

import jax
import jax.numpy as jnp
from jax import lax
from jax.experimental import pallas as pl
from jax.experimental.pallas import tpu as pltpu
from jax.experimental.pallas import tpu_sc as plsc

N_IN = 10000
UP = 8
N_UP = N_IN * UP
E_UP = 1280000
C_IN = 64
C_HID = 64
C_OUT = 32
HALF = 16
L_BLOCK = 3

G_HID = UP * C_HID
G_OUT = UP * C_OUT
G_HALF = UP * HALF

NS = 16
K_CH = 1000

_R = 2000
_NG = N_IN // _R


def _up_body(x_ref, w_ref, b_ref, o_ref):
    o_ref[...] = jnp.maximum(
        jnp.dot(x_ref[...], w_ref[...], preferred_element_type=jnp.float32)
        + b_ref[...], 0.0)


def _up(x, wf, bf):
    R = 1000
    return pl.pallas_call(
        _up_body,
        grid=(N_IN // R,),
        in_specs=[pl.BlockSpec((R, C_IN), lambda i: (i, 0)),
                  pl.BlockSpec((C_IN, G_HID), lambda i: (0, 0)),
                  pl.BlockSpec((1, G_HID), lambda i: (0, 0))],
        out_specs=pl.BlockSpec((R, G_HID), lambda i: (i, 0)),
        out_shape=jax.ShapeDtypeStruct((N_IN, G_HID), jnp.float32),
    )(x, wf, bf)


def _proj0_body(h_ref, ws_ref, wlo_ref, whi_ref, b_ref,
                s_ref, tlo_ref, thi_ref):
    hh = h_ref[...]
    s_ref[...] = (jnp.dot(hh, ws_ref[...], preferred_element_type=jnp.float32)
                  + b_ref[...])
    tlo_ref[...] = jnp.dot(hh, wlo_ref[...], preferred_element_type=jnp.float32)
    thi_ref[...] = jnp.dot(hh, whi_ref[...], preferred_element_type=jnp.float32)


def _proj0(h, ws, wlo, whi, b):
    C = h.shape[1]
    return pl.pallas_call(
        _proj0_body,
        grid=(_NG,),
        in_specs=[pl.BlockSpec((_R, C), lambda i: (i, 0)),
                  pl.BlockSpec((C, G_OUT), lambda i: (0, 0)),
                  pl.BlockSpec((C, G_HALF), lambda i: (0, 0)),
                  pl.BlockSpec((C, G_HALF), lambda i: (0, 0)),
                  pl.BlockSpec((1, G_OUT), lambda i: (0, 0))],
        out_specs=[pl.BlockSpec((_R, G_OUT), lambda i: (i, 0)),
                   pl.BlockSpec((_R, G_HALF), lambda i: (i, 0)),
                   pl.BlockSpec((_R, G_HALF), lambda i: (i, 0))],
        out_shape=[jax.ShapeDtypeStruct((N_IN, G_OUT), jnp.float32),
                   jax.ShapeDtypeStruct((N_IN, G_HALF), jnp.float32),
                   jax.ShapeDtypeStruct((N_IN, G_HALF), jnp.float32)],
    )(h, ws, wlo, whi, b)


def _cproj_body(s_ref, ylo_ref, yhi_ref, plo_ref, phi_ref,
                ws_ref, wlo_ref, whi_ref, b_ref,
                h_ref, s2_ref, tlo_ref, thi_ref):
    y = (jnp.dot(ylo_ref[...], plo_ref[...], preferred_element_type=jnp.float32)
         + jnp.dot(yhi_ref[...], phi_ref[...], preferred_element_type=jnp.float32))
    h = jnp.maximum(s_ref[...] + y, 0.0)
    h_ref[...] = h
    s2_ref[...] = (jnp.dot(h, ws_ref[...], preferred_element_type=jnp.float32)
                   + b_ref[...])
    tlo_ref[...] = jnp.dot(h, wlo_ref[...], preferred_element_type=jnp.float32)
    thi_ref[...] = jnp.dot(h, whi_ref[...], preferred_element_type=jnp.float32)


def _cproj_res_body(res_ref, s_ref, ylo_ref, yhi_ref, plo_ref, phi_ref,
                    ws_ref, wlo_ref, whi_ref, b_ref,
                    h_ref, s2_ref, tlo_ref, thi_ref):
    y = (jnp.dot(ylo_ref[...], plo_ref[...], preferred_element_type=jnp.float32)
         + jnp.dot(yhi_ref[...], phi_ref[...], preferred_element_type=jnp.float32))
    h = jnp.maximum(res_ref[...] + s_ref[...] + y, 0.0)
    h_ref[...] = h
    s2_ref[...] = (jnp.dot(h, ws_ref[...], preferred_element_type=jnp.float32)
                   + b_ref[...])
    tlo_ref[...] = jnp.dot(h, wlo_ref[...], preferred_element_type=jnp.float32)
    thi_ref[...] = jnp.dot(h, whi_ref[...], preferred_element_type=jnp.float32)


def _row_spec(w):
    return pl.BlockSpec((_R, w), lambda i: (i, 0))


def _full_spec(r, w):
    return pl.BlockSpec((r, w), lambda i: (0, 0))


_CPROJ_OUT = [jax.ShapeDtypeStruct((N_IN, G_OUT), jnp.float32),
              jax.ShapeDtypeStruct((N_IN, G_OUT), jnp.float32),
              jax.ShapeDtypeStruct((N_IN, G_HALF), jnp.float32),
              jax.ShapeDtypeStruct((N_IN, G_HALF), jnp.float32)]

_CPROJ_OUT_SPECS = [pl.BlockSpec((_R, G_OUT), lambda i: (i, 0)),
                    pl.BlockSpec((_R, G_OUT), lambda i: (i, 0)),
                    pl.BlockSpec((_R, G_HALF), lambda i: (i, 0)),
                    pl.BlockSpec((_R, G_HALF), lambda i: (i, 0))]


def _cproj(s, ylo, yhi, plo, phi, ws, wlo, whi, b):
    return pl.pallas_call(
        _cproj_body,
        grid=(_NG,),
        in_specs=[_row_spec(G_OUT), _row_spec(G_HALF), _row_spec(G_HALF),
                  _full_spec(G_HALF, G_OUT), _full_spec(G_HALF, G_OUT),
                  _full_spec(G_OUT, G_OUT), _full_spec(G_OUT, G_HALF),
                  _full_spec(G_OUT, G_HALF), _full_spec(1, G_OUT)],
        out_specs=_CPROJ_OUT_SPECS,
        out_shape=_CPROJ_OUT,
    )(s, ylo, yhi, plo, phi, ws, wlo, whi, b)


def _cproj_res(res, s, ylo, yhi, plo, phi, ws, wlo, whi, b):
    return pl.pallas_call(
        _cproj_res_body,
        grid=(_NG,),
        in_specs=[_row_spec(G_OUT),
                  _row_spec(G_OUT), _row_spec(G_HALF), _row_spec(G_HALF),
                  _full_spec(G_HALF, G_OUT), _full_spec(G_HALF, G_OUT),
                  _full_spec(G_OUT, G_OUT), _full_spec(G_OUT, G_HALF),
                  _full_spec(G_OUT, G_HALF), _full_spec(1, G_OUT)],
        out_specs=_CPROJ_OUT_SPECS,
        out_shape=_CPROJ_OUT,
    )(res, s, ylo, yhi, plo, phi, ws, wlo, whi, b)


def _ccls_body(res_ref, s_ref, ylo_ref, yhi_ref, plo_ref, phi_ref,
               wcs_ref, wcn_ref, bc_ref,
               h_ref, sc_ref, tpad_ref):
    y = (jnp.dot(ylo_ref[...], plo_ref[...], preferred_element_type=jnp.float32)
         + jnp.dot(yhi_ref[...], phi_ref[...], preferred_element_type=jnp.float32))
    h = jnp.maximum(res_ref[...] + s_ref[...] + y, 0.0)
    h_ref[...] = h
    sc_ref[...] = (jnp.dot(h, wcs_ref[...], preferred_element_type=jnp.float32)
                   + bc_ref[...])
    tpad_ref[...] = jnp.dot(h, wcn_ref[...], preferred_element_type=jnp.float32)


def _ccls(res, s, ylo, yhi, plo, phi, wcs, wcn, bc):
    return pl.pallas_call(
        _ccls_body,
        grid=(_NG,),
        in_specs=[_row_spec(G_OUT),
                  _row_spec(G_OUT), _row_spec(G_HALF), _row_spec(G_HALF),
                  _full_spec(G_HALF, G_OUT), _full_spec(G_HALF, G_OUT),
                  _full_spec(G_OUT, G_HALF), _full_spec(G_OUT, G_HALF),
                  _full_spec(1, G_HALF)],
        out_specs=[pl.BlockSpec((_R, G_OUT), lambda i: (i, 0)),
                   pl.BlockSpec((_R, G_HALF), lambda i: (i, 0)),
                   pl.BlockSpec((_R, G_HALF), lambda i: (i, 0))],
        out_shape=[jax.ShapeDtypeStruct((N_IN, G_OUT), jnp.float32),
                   jax.ShapeDtypeStruct((N_IN, G_HALF), jnp.float32),
                   jax.ShapeDtypeStruct((N_IN, G_HALF), jnp.float32)],
    )(res, s, ylo, yhi, plo, phi, wcs, wcn, bc)


def _finreduce_body(sc_ref, y0_ref, y1_ref, lg_ref, mx_ref, top_ref):
    lg = sc_ref[...] + y0_ref[...] + y1_ref[...]
    lg_ref[...] = lg
    col = lax.broadcasted_iota(jnp.int32, (N_IN, G_HALF), 1)
    node = (lax.broadcasted_iota(jnp.int32, (N_IN, G_HALF), 0) * UP
            + col // HALF)
    valid = (col % HALF) == 0
    neg = jnp.float32(-3.0e38)
    mx = jnp.max(jnp.where(valid, lg, neg))
    mx_ref[...] = jnp.reshape(mx, (1, 1))
    top_ref[...] = jnp.reshape(
        jnp.min(jnp.where(valid & (lg == mx), node, jnp.int32(2**30))), (1, 1))


def _finreduce(sc, y0, y1):
    return pl.pallas_call(
        _finreduce_body,
        in_specs=[pl.BlockSpec((N_IN, G_HALF), lambda: (0, 0))] * 3,
        out_specs=[pl.BlockSpec((N_IN, G_HALF), lambda: (0, 0)),
                   pl.BlockSpec((1, 1), lambda: (0, 0)),
                   pl.BlockSpec((1, 1), lambda: (0, 0))],
        out_shape=[jax.ShapeDtypeStruct((N_IN, G_HALF), jnp.float32),
                   jax.ShapeDtypeStruct((1, 1), jnp.float32),
                   jax.ShapeDtypeStruct((1, 1), jnp.int32)],
    )(sc, y0, y1)


def _mask_body(h_ref, lg_ref, tgt_ref, mx_ref, top_ref, o_ref, k_ref):
    R = 4000
    pid = pl.program_id(0)
    lg = lg_ref[...]
    iota = lax.broadcasted_iota(jnp.int32, (R, 1), 0) + pid * R
    keep = ((lg > 0.0) | (tgt_ref[...] != 0)
            | ((iota == top_ref[0, 0]) & (mx_ref[0, 0] < 0.0)))
    o_ref[...] = h_ref[...] * keep.astype(jnp.float32)
    k_ref[...] = keep.astype(jnp.int32)


def _mask(h, lgN, tgtN, mx, top):
    R = 4000
    return pl.pallas_call(
        _mask_body,
        grid=(N_UP // R,),
        in_specs=[pl.BlockSpec((R, C_OUT), lambda i: (i, 0)),
                  pl.BlockSpec((R, 1), lambda i: (i, 0)),
                  pl.BlockSpec((R, 1), lambda i: (i, 0)),
                  pl.BlockSpec((1, 1), lambda i: (0, 0)),
                  pl.BlockSpec((1, 1), lambda i: (0, 0))],
        out_specs=[pl.BlockSpec((R, C_OUT), lambda i: (i, 0)),
                   pl.BlockSpec((R, 1), lambda i: (i, 0))],
        out_shape=[jax.ShapeDtypeStruct((N_UP, C_OUT), jnp.float32),
                   jax.ShapeDtypeStruct((N_UP, 1), jnp.int32)],
    )(h, lgN, tgtN, mx, top)


_MESH = plsc.VectorSubcoreMesh(core_axis_name="c", subcore_axis_name="s",
                               num_cores=2, num_subcores=NS)

_SC_SCRATCH = [
    pltpu.VMEM((K_CH,), jnp.int32),
    pltpu.VMEM((K_CH,), jnp.int32),
    pltpu.VMEM((K_CH, HALF), jnp.float32),
    pltpu.VMEM((K_CH,), jnp.int32),
    pltpu.VMEM((K_CH,), jnp.int32),
    pltpu.VMEM((K_CH, HALF), jnp.float32),
    pltpu.VMEM_SHARED((N_UP, HALF), jnp.float32),
    pltpu.SemaphoreType.DMA,
    pltpu.SemaphoreType.DMA,
]


def _fetch(table_h, src_h, dst_h, buf, base):
    srcv, dstv, rows, sem = buf
    pltpu.sync_copy(src_h.at[pl.ds(base, K_CH)], srcv)
    pltpu.sync_copy(dst_h.at[pl.ds(base, K_CH)], dstv)
    pltpu.async_copy(table_h.at[srcv], rows, sem)


def _drain(table_h, acc, buf):
    srcv, dstv, rows, sem = buf
    pltpu.make_async_copy(table_h.at[srcv], rows, sem).wait()
    pltpu.sync_copy(rows, acc.at[dstv], add=True)


def _edge_loop(table_h, src_h, dst_h, acc, buf0, buf1, base0, nch):
    def body2(i, carry):
        g0 = 2 * i
        _fetch(table_h, src_h, dst_h, buf1, base0 + (g0 + 1) * K_CH)
        _drain(table_h, acc, buf0)

        @pl.when(g0 + 2 < nch)
        def _():
            _fetch(table_h, src_h, dst_h, buf0, base0 + (g0 + 2) * K_CH)

        _drain(table_h, acc, buf1)
        return carry
    lax.fori_loop(0, nch // 2, body2, 0)


@functools.partial(
    pl.kernel,
    out_type=[jax.ShapeDtypeStruct((N_UP, HALF), jnp.float32)] * 2,
    mesh=_MESH,
    scratch_types=_SC_SCRATCH,
    compiler_params=pltpu.CompilerParams(use_tc_tiling_on_sc=False),
)
def _sc_wide(tlo_h, thi_h, src_h, dst_h, zer_h, ylo_h, yhi_h,
             srcv0, dstv0, rows0, srcv1, dstv1, rows1, acc, sem0, sem1):
    cid = lax.axis_index("c")
    sid = lax.axis_index("s")
    buf0 = (srcv0, dstv0, rows0, sem0)
    buf1 = (srcv1, dstv1, rows1, sem1)
    rpt = N_UP // NS
    row0 = sid * rpt
    ept = E_UP // NS
    nch = ept // K_CH

    def run(table_h):
        _fetch(table_h, src_h, dst_h, buf0, sid * ept)
        pltpu.sync_copy(zer_h.at[pl.ds(row0, rpt)], acc.at[pl.ds(row0, rpt)])
        plsc.subcore_barrier()
        _edge_loop(table_h, src_h, dst_h, acc, buf0, buf1, sid * ept, nch)

    @pl.when(cid == 0)
    def _():
        run(tlo_h)

    @pl.when(cid == 1)
    def _():
        run(thi_h)

    plsc.subcore_barrier()

    @pl.when(cid == 0)
    def _():
        pltpu.sync_copy(acc.at[pl.ds(row0, rpt)], ylo_h.at[pl.ds(row0, rpt)])

    @pl.when(cid == 1)
    def _():
        pltpu.sync_copy(acc.at[pl.ds(row0, rpt)], yhi_h.at[pl.ds(row0, rpt)])


@functools.partial(
    pl.kernel,
    out_type=[jax.ShapeDtypeStruct((N_UP, HALF), jnp.float32)] * 2,
    mesh=_MESH,
    scratch_types=_SC_SCRATCH,
    compiler_params=pltpu.CompilerParams(use_tc_tiling_on_sc=False),
)
def _sc_cls(t_h, src_h, dst_h, zer_h, y0_h, y1_h,
            srcv0, dstv0, rows0, srcv1, dstv1, rows1, acc, sem0, sem1):
    cid = lax.axis_index("c")
    sid = lax.axis_index("s")
    buf0 = (srcv0, dstv0, rows0, sem0)
    buf1 = (srcv1, dstv1, rows1, sem1)
    rpt = N_UP // NS
    row0 = sid * rpt
    ept = E_UP // (2 * NS)
    nch = ept // K_CH
    base0 = cid * (E_UP // 2) + sid * ept
    _fetch(t_h, src_h, dst_h, buf0, base0)
    pltpu.sync_copy(zer_h.at[pl.ds(row0, rpt)], acc.at[pl.ds(row0, rpt)])
    plsc.subcore_barrier()
    _edge_loop(t_h, src_h, dst_h, acc, buf0, buf1, base0, nch)
    plsc.subcore_barrier()

    @pl.when(cid == 0)
    def _():
        pltpu.sync_copy(acc.at[pl.ds(row0, rpt)], y0_h.at[pl.ds(row0, rpt)])

    @pl.when(cid == 1)
    def _():
        pltpu.sync_copy(acc.at[pl.ds(row0, rpt)], y1_h.at[pl.ds(row0, rpt)])


def _bd8(w):
    c, d = w.shape
    return jnp.einsum('jk,cd->jckd', jnp.eye(UP, dtype=w.dtype), w).reshape(
        UP * c, UP * d)


def _grouped(a):
    return a.reshape(N_IN, G_HALF)


def _flat16(a):
    return a.reshape(N_UP, HALF)


def kernel(x, edge_index_up, target_label, W_up, b_up, W1_self, W1_nbr, b1,
           Wb_self, Wb_nbr, bb, Wc_self, Wc_nbr, bc):
    f32 = jnp.float32
    src = edge_index_up[0]
    dst = edge_index_up[1]
    zer = jnp.zeros((N_UP, HALF), f32)

    plo = _bd8(jnp.concatenate(
        [jnp.eye(HALF, dtype=f32), jnp.zeros((HALF, HALF), f32)], axis=1))
    phi = _bd8(jnp.concatenate(
        [jnp.zeros((HALF, HALF), f32), jnp.eye(HALF, dtype=f32)], axis=1))

    wf_up = jnp.transpose(W_up, (1, 0, 2)).reshape(C_IN, G_HID)
    bf_up = jnp.tile(b_up, UP).reshape(1, G_HID)
    h = _up(x, wf_up, bf_up)

    s, tlo, thi = _proj0(h, _bd8(W1_self), _bd8(W1_nbr[:, :HALF]),
                         _bd8(W1_nbr[:, HALF:]),
                         jnp.tile(b1, UP).reshape(1, G_OUT))
    ylo, yhi = _sc_wide(_flat16(tlo), _flat16(thi), src, dst, zer)

    hres, s, tlo, thi = _cproj(
        s, _grouped(ylo), _grouped(yhi), plo, phi,
        _bd8(Wb_self[0, 0]), _bd8(Wb_nbr[0, 0][:, :HALF]),
        _bd8(Wb_nbr[0, 0][:, HALF:]), jnp.tile(bb[0, 0], UP).reshape(1, G_OUT))
    ylo, yhi = _sc_wide(_flat16(tlo), _flat16(thi), src, dst, zer)

    for l in range(L_BLOCK):
        _, s, tlo, thi = _cproj(
            s, _grouped(ylo), _grouped(yhi), plo, phi,
            _bd8(Wb_self[l, 1]), _bd8(Wb_nbr[l, 1][:, :HALF]),
            _bd8(Wb_nbr[l, 1][:, HALF:]),
            jnp.tile(bb[l, 1], UP).reshape(1, G_OUT))
        ylo, yhi = _sc_wide(_flat16(tlo), _flat16(thi), src, dst, zer)
        if l < L_BLOCK - 1:
            hres, s, tlo, thi = _cproj_res(
                hres, s, _grouped(ylo), _grouped(yhi), plo, phi,
                _bd8(Wb_self[l + 1, 0]), _bd8(Wb_nbr[l + 1, 0][:, :HALF]),
                _bd8(Wb_nbr[l + 1, 0][:, HALF:]),
                jnp.tile(bb[l + 1, 0], UP).reshape(1, G_OUT))
            ylo, yhi = _sc_wide(_flat16(tlo), _flat16(thi), src, dst, zer)

    e0 = jnp.zeros((1, HALF), f32).at[0, 0].set(1.0)
    wcs = _bd8(Wc_self @ e0)
    wcn = _bd8(Wc_nbr @ e0)
    bc_g = jnp.tile(bc[0] * e0, (1, UP))
    hfin, sc_g, tpad = _ccls(hres, s, _grouped(ylo), _grouped(yhi), plo, phi,
                             wcs, wcn, bc_g)
    y0, y1 = _sc_cls(_flat16(tpad), src, dst, zer)

    lg_g, mx, top = _finreduce(sc_g, _grouped(y0), _grouped(y1))

    out_cls = lg_g.reshape(N_UP, HALF)[:, :1]
    hN = hfin.reshape(N_UP, C_OUT)
    tgtN = target_label.astype(jnp.int32).reshape(N_UP, 1)
    out_pruned, keep_i = _mask(hN, out_cls, tgtN, mx, top)
    keep = keep_i.reshape(N_UP) != 0
    return out_pruned, out_cls, target_label, keep


# speedup vs baseline: 41.3549x; 1.1447x over previous
import functools

import jax
import jax.numpy as jnp
from jax import lax
from jax.experimental import pallas as pl
from jax.experimental.pallas import tpu as pltpu
from jax.experimental.pallas import tpu_sc as plsc

N_IN = 10000
UP = 8
N_UP = N_IN * UP
E_UP = 1280000
C_IN = 64
C_HID = 64
C_OUT = 32
HALF = 16
L_BLOCK = 3

G_HID = UP * C_HID
G_OUT = UP * C_OUT
G_HALF = UP * HALF

NS = 16
K_CH = 400
NSLOT = 5

_R = 2000
_NG = N_IN // _R


def _up_body(x_ref, w_ref, b_ref, o_ref):
    o_ref[...] = jnp.maximum(
        jnp.dot(x_ref[...], w_ref[...], preferred_element_type=jnp.float32)
        + b_ref[...], 0.0)


def _up(x, wf, bf):
    R = 1000
    return pl.pallas_call(
        _up_body,
        grid=(N_IN // R,),
        in_specs=[pl.BlockSpec((R, C_IN), lambda i: (i, 0)),
                  pl.BlockSpec((C_IN, G_HID), lambda i: (0, 0)),
                  pl.BlockSpec((1, G_HID), lambda i: (0, 0))],
        out_specs=pl.BlockSpec((R, G_HID), lambda i: (i, 0)),
        out_shape=jax.ShapeDtypeStruct((N_IN, G_HID), jnp.float32),
    )(x, wf, bf)


def _proj0_body(h_ref, ws_ref, wlo_ref, whi_ref, b_ref,
                s_ref, tlo_ref, thi_ref):
    hh = h_ref[...]
    s_ref[...] = (jnp.dot(hh, ws_ref[...], preferred_element_type=jnp.float32)
                  + b_ref[...])
    tlo_ref[...] = jnp.dot(hh, wlo_ref[...], preferred_element_type=jnp.float32)
    thi_ref[...] = jnp.dot(hh, whi_ref[...], preferred_element_type=jnp.float32)


def _proj0(h, ws, wlo, whi, b):
    C = h.shape[1]
    return pl.pallas_call(
        _proj0_body,
        grid=(_NG,),
        in_specs=[pl.BlockSpec((_R, C), lambda i: (i, 0)),
                  pl.BlockSpec((C, G_OUT), lambda i: (0, 0)),
                  pl.BlockSpec((C, G_HALF), lambda i: (0, 0)),
                  pl.BlockSpec((C, G_HALF), lambda i: (0, 0)),
                  pl.BlockSpec((1, G_OUT), lambda i: (0, 0))],
        out_specs=[pl.BlockSpec((_R, G_OUT), lambda i: (i, 0)),
                   pl.BlockSpec((_R, G_HALF), lambda i: (i, 0)),
                   pl.BlockSpec((_R, G_HALF), lambda i: (i, 0))],
        out_shape=[jax.ShapeDtypeStruct((N_IN, G_OUT), jnp.float32),
                   jax.ShapeDtypeStruct((N_IN, G_HALF), jnp.float32),
                   jax.ShapeDtypeStruct((N_IN, G_HALF), jnp.float32)],
    )(h, ws, wlo, whi, b)


def _cproj_body(s_ref, ylo_ref, yhi_ref, plo_ref, phi_ref,
                ws_ref, wlo_ref, whi_ref, b_ref,
                h_ref, s2_ref, tlo_ref, thi_ref):
    y = (jnp.dot(ylo_ref[...], plo_ref[...], preferred_element_type=jnp.float32)
         + jnp.dot(yhi_ref[...], phi_ref[...], preferred_element_type=jnp.float32))
    h = jnp.maximum(s_ref[...] + y, 0.0)
    h_ref[...] = h
    s2_ref[...] = (jnp.dot(h, ws_ref[...], preferred_element_type=jnp.float32)
                   + b_ref[...])
    tlo_ref[...] = jnp.dot(h, wlo_ref[...], preferred_element_type=jnp.float32)
    thi_ref[...] = jnp.dot(h, whi_ref[...], preferred_element_type=jnp.float32)


def _cproj_res_body(res_ref, s_ref, ylo_ref, yhi_ref, plo_ref, phi_ref,
                    ws_ref, wlo_ref, whi_ref, b_ref,
                    h_ref, s2_ref, tlo_ref, thi_ref):
    y = (jnp.dot(ylo_ref[...], plo_ref[...], preferred_element_type=jnp.float32)
         + jnp.dot(yhi_ref[...], phi_ref[...], preferred_element_type=jnp.float32))
    h = jnp.maximum(res_ref[...] + s_ref[...] + y, 0.0)
    h_ref[...] = h
    s2_ref[...] = (jnp.dot(h, ws_ref[...], preferred_element_type=jnp.float32)
                   + b_ref[...])
    tlo_ref[...] = jnp.dot(h, wlo_ref[...], preferred_element_type=jnp.float32)
    thi_ref[...] = jnp.dot(h, whi_ref[...], preferred_element_type=jnp.float32)


def _row_spec(w):
    return pl.BlockSpec((_R, w), lambda i: (i, 0))


def _full_spec(r, w):
    return pl.BlockSpec((r, w), lambda i: (0, 0))


_CPROJ_OUT = [jax.ShapeDtypeStruct((N_IN, G_OUT), jnp.float32),
              jax.ShapeDtypeStruct((N_IN, G_OUT), jnp.float32),
              jax.ShapeDtypeStruct((N_IN, G_HALF), jnp.float32),
              jax.ShapeDtypeStruct((N_IN, G_HALF), jnp.float32)]

_CPROJ_OUT_SPECS = [pl.BlockSpec((_R, G_OUT), lambda i: (i, 0)),
                    pl.BlockSpec((_R, G_OUT), lambda i: (i, 0)),
                    pl.BlockSpec((_R, G_HALF), lambda i: (i, 0)),
                    pl.BlockSpec((_R, G_HALF), lambda i: (i, 0))]


def _cproj(s, ylo, yhi, plo, phi, ws, wlo, whi, b):
    return pl.pallas_call(
        _cproj_body,
        grid=(_NG,),
        in_specs=[_row_spec(G_OUT), _row_spec(G_HALF), _row_spec(G_HALF),
                  _full_spec(G_HALF, G_OUT), _full_spec(G_HALF, G_OUT),
                  _full_spec(G_OUT, G_OUT), _full_spec(G_OUT, G_HALF),
                  _full_spec(G_OUT, G_HALF), _full_spec(1, G_OUT)],
        out_specs=_CPROJ_OUT_SPECS,
        out_shape=_CPROJ_OUT,
    )(s, ylo, yhi, plo, phi, ws, wlo, whi, b)


def _cproj_res(res, s, ylo, yhi, plo, phi, ws, wlo, whi, b):
    return pl.pallas_call(
        _cproj_res_body,
        grid=(_NG,),
        in_specs=[_row_spec(G_OUT),
                  _row_spec(G_OUT), _row_spec(G_HALF), _row_spec(G_HALF),
                  _full_spec(G_HALF, G_OUT), _full_spec(G_HALF, G_OUT),
                  _full_spec(G_OUT, G_OUT), _full_spec(G_OUT, G_HALF),
                  _full_spec(G_OUT, G_HALF), _full_spec(1, G_OUT)],
        out_specs=_CPROJ_OUT_SPECS,
        out_shape=_CPROJ_OUT,
    )(res, s, ylo, yhi, plo, phi, ws, wlo, whi, b)


def _ccls_body(res_ref, s_ref, ylo_ref, yhi_ref, plo_ref, phi_ref,
               wcs_ref, wcn_ref, bc_ref,
               h_ref, sc_ref, tpad_ref):
    y = (jnp.dot(ylo_ref[...], plo_ref[...], preferred_element_type=jnp.float32)
         + jnp.dot(yhi_ref[...], phi_ref[...], preferred_element_type=jnp.float32))
    h = jnp.maximum(res_ref[...] + s_ref[...] + y, 0.0)
    h_ref[...] = h
    sc_ref[...] = (jnp.dot(h, wcs_ref[...], preferred_element_type=jnp.float32)
                   + bc_ref[...])
    tpad_ref[...] = jnp.dot(h, wcn_ref[...], preferred_element_type=jnp.float32)


def _ccls(res, s, ylo, yhi, plo, phi, wcs, wcn, bc):
    return pl.pallas_call(
        _ccls_body,
        grid=(_NG,),
        in_specs=[_row_spec(G_OUT),
                  _row_spec(G_OUT), _row_spec(G_HALF), _row_spec(G_HALF),
                  _full_spec(G_HALF, G_OUT), _full_spec(G_HALF, G_OUT),
                  _full_spec(G_OUT, G_HALF), _full_spec(G_OUT, G_HALF),
                  _full_spec(1, G_HALF)],
        out_specs=[pl.BlockSpec((_R, G_OUT), lambda i: (i, 0)),
                   pl.BlockSpec((_R, G_HALF), lambda i: (i, 0)),
                   pl.BlockSpec((_R, G_HALF), lambda i: (i, 0))],
        out_shape=[jax.ShapeDtypeStruct((N_IN, G_OUT), jnp.float32),
                   jax.ShapeDtypeStruct((N_IN, G_HALF), jnp.float32),
                   jax.ShapeDtypeStruct((N_IN, G_HALF), jnp.float32)],
    )(res, s, ylo, yhi, plo, phi, wcs, wcn, bc)


def _finreduce_body(sc_ref, y0_ref, y1_ref, lg_ref, mx_ref, top_ref):
    lg = sc_ref[...] + y0_ref[...] + y1_ref[...]
    lg_ref[...] = lg
    col = lax.broadcasted_iota(jnp.int32, (N_IN, G_HALF), 1)
    node = (lax.broadcasted_iota(jnp.int32, (N_IN, G_HALF), 0) * UP
            + col // HALF)
    valid = (col % HALF) == 0
    neg = jnp.float32(-3.0e38)
    mx = jnp.max(jnp.where(valid, lg, neg))
    mx_ref[...] = jnp.reshape(mx, (1, 1))
    top_ref[...] = jnp.reshape(
        jnp.min(jnp.where(valid & (lg == mx), node, jnp.int32(2**30))), (1, 1))


def _finreduce(sc, y0, y1):
    return pl.pallas_call(
        _finreduce_body,
        in_specs=[pl.BlockSpec((N_IN, G_HALF), lambda: (0, 0))] * 3,
        out_specs=[pl.BlockSpec((N_IN, G_HALF), lambda: (0, 0)),
                   pl.BlockSpec((1, 1), lambda: (0, 0)),
                   pl.BlockSpec((1, 1), lambda: (0, 0))],
        out_shape=[jax.ShapeDtypeStruct((N_IN, G_HALF), jnp.float32),
                   jax.ShapeDtypeStruct((1, 1), jnp.float32),
                   jax.ShapeDtypeStruct((1, 1), jnp.int32)],
    )(sc, y0, y1)


def _mask_body(h_ref, lg_ref, tgt_ref, mx_ref, top_ref, o_ref, k_ref):
    R = 4000
    pid = pl.program_id(0)
    lg = lg_ref[...]
    iota = lax.broadcasted_iota(jnp.int32, (R, 1), 0) + pid * R
    keep = ((lg > 0.0) | (tgt_ref[...] != 0)
            | ((iota == top_ref[0, 0]) & (mx_ref[0, 0] < 0.0)))
    o_ref[...] = h_ref[...] * keep.astype(jnp.float32)
    k_ref[...] = keep.astype(jnp.int32)


def _mask(h, lgN, tgtN, mx, top):
    R = 4000
    return pl.pallas_call(
        _mask_body,
        grid=(N_UP // R,),
        in_specs=[pl.BlockSpec((R, C_OUT), lambda i: (i, 0)),
                  pl.BlockSpec((R, 1), lambda i: (i, 0)),
                  pl.BlockSpec((R, 1), lambda i: (i, 0)),
                  pl.BlockSpec((1, 1), lambda i: (0, 0)),
                  pl.BlockSpec((1, 1), lambda i: (0, 0))],
        out_specs=[pl.BlockSpec((R, C_OUT), lambda i: (i, 0)),
                   pl.BlockSpec((R, 1), lambda i: (i, 0))],
        out_shape=[jax.ShapeDtypeStruct((N_UP, C_OUT), jnp.float32),
                   jax.ShapeDtypeStruct((N_UP, 1), jnp.int32)],
    )(h, lgN, tgtN, mx, top)


_MESH = plsc.VectorSubcoreMesh(core_axis_name="c", subcore_axis_name="s",
                               num_cores=2, num_subcores=NS)

_SC_SCRATCH = (
    [pltpu.VMEM((K_CH,), jnp.int32) for _ in range(NSLOT)]
    + [pltpu.VMEM((K_CH,), jnp.int32) for _ in range(NSLOT)]
    + [pltpu.VMEM((K_CH, HALF), jnp.float32) for _ in range(NSLOT)]
    + [pltpu.VMEM_SHARED((N_UP, HALF), jnp.float32)]
    + [pltpu.SemaphoreType.DMA] * (3 * NSLOT)
)


def _mk_slots(scr):
    srcvs = scr[0:NSLOT]
    dstvs = scr[NSLOT:2 * NSLOT]
    rows = scr[2 * NSLOT:3 * NSLOT]
    sems = scr[3 * NSLOT + 1:]
    isems, gsems, ssems = (sems[0:NSLOT], sems[NSLOT:2 * NSLOT],
                           sems[2 * NSLOT:3 * NSLOT])
    return [(srcvs[s], dstvs[s], rows[s], isems[s], gsems[s], ssems[s])
            for s in range(NSLOT)]


def _edge_loop(table_h, src_h, dst_h, acc, slots, base0, nch):
    def idx_copies(slot, c):
        srcv, dstv, _, isem, _, _ = slots[slot]
        base = base0 + c * K_CH
        return (pltpu.make_async_copy(src_h.at[pl.ds(base, K_CH)], srcv, isem),
                pltpu.make_async_copy(dst_h.at[pl.ds(base, K_CH)], dstv, isem))

    def stage(c_idx, s_idx, c_gat, s_gat, c_sct, s_sct):
        srcv, dstv, rows, isem, gsem, ssem = slots[s_idx]

        @pl.when((c_idx < nch) & (c_idx >= NSLOT))
        def _():
            pltpu.make_async_copy(rows, acc.at[dstv], ssem).wait()

        @pl.when(c_idx < nch)
        def _():
            for d in idx_copies(s_idx, c_idx):
                d.start()

        srcvg, dstvg, rowsg, isemg, gsemg, ssemg = slots[s_gat]

        @pl.when((c_gat >= 0) & (c_gat < nch))
        def _():
            for d in idx_copies(s_gat, c_gat):
                d.wait()
            pltpu.async_copy(table_h.at[srcvg], rowsg, gsemg)

        srcvs, dstvs, rowss, isems, gsems, ssems = slots[s_sct]

        @pl.when((c_sct >= 0) & (c_sct < nch))
        def _():
            pltpu.make_async_copy(table_h.at[srcvs], rowss, gsems).wait()
            pltpu.async_copy(rowss, acc.at[dstvs], ssems, add=True)

    nsup = (nch + 2 + NSLOT - 1) // NSLOT

    def sup(i, carry):
        g0 = i * NSLOT
        for j in range(NSLOT):
            g = g0 + j
            stage(g, j, g - 1, (j - 1) % NSLOT, g - 2, (j - 2) % NSLOT)
        return carry
    lax.fori_loop(0, nsup, sup, 0)
    for c in range(max(0, nch - NSLOT), nch):
        _, dstvs, rowss, _, _, ssems = slots[c % NSLOT]
        pltpu.make_async_copy(rowss, acc.at[dstvs], ssems).wait()


@functools.partial(
    pl.kernel,
    out_type=[jax.ShapeDtypeStruct((N_UP, HALF), jnp.float32)] * 2,
    mesh=_MESH,
    scratch_types=_SC_SCRATCH,
    compiler_params=pltpu.CompilerParams(use_tc_tiling_on_sc=False),
)
def _sc_wide(tlo_h, thi_h, src_h, dst_h, zer_h, ylo_h, yhi_h, *scr):
    cid = lax.axis_index("c")
    sid = lax.axis_index("s")
    slots = _mk_slots(scr)
    acc = scr[3 * NSLOT]
    rpt = N_UP // NS
    row0 = sid * rpt
    ept = E_UP // NS
    nch = ept // K_CH
    pltpu.sync_copy(zer_h.at[pl.ds(row0, rpt)], acc.at[pl.ds(row0, rpt)])
    plsc.subcore_barrier()

    @pl.when(cid == 0)
    def _():
        _edge_loop(tlo_h, src_h, dst_h, acc, slots, sid * ept, nch)

    @pl.when(cid == 1)
    def _():
        _edge_loop(thi_h, src_h, dst_h, acc, slots, sid * ept, nch)

    plsc.subcore_barrier()

    @pl.when(cid == 0)
    def _():
        pltpu.sync_copy(acc.at[pl.ds(row0, rpt)], ylo_h.at[pl.ds(row0, rpt)])

    @pl.when(cid == 1)
    def _():
        pltpu.sync_copy(acc.at[pl.ds(row0, rpt)], yhi_h.at[pl.ds(row0, rpt)])


@functools.partial(
    pl.kernel,
    out_type=[jax.ShapeDtypeStruct((N_UP, HALF), jnp.float32)] * 2,
    mesh=_MESH,
    scratch_types=_SC_SCRATCH,
    compiler_params=pltpu.CompilerParams(use_tc_tiling_on_sc=False),
)
def _sc_cls(t_h, src_h, dst_h, zer_h, y0_h, y1_h, *scr):
    cid = lax.axis_index("c")
    sid = lax.axis_index("s")
    slots = _mk_slots(scr)
    acc = scr[3 * NSLOT]
    rpt = N_UP // NS
    row0 = sid * rpt
    ept = E_UP // (2 * NS)
    nch = ept // K_CH
    base0 = cid * (E_UP // 2) + sid * ept
    pltpu.sync_copy(zer_h.at[pl.ds(row0, rpt)], acc.at[pl.ds(row0, rpt)])
    plsc.subcore_barrier()
    _edge_loop(t_h, src_h, dst_h, acc, slots, base0, nch)
    plsc.subcore_barrier()

    @pl.when(cid == 0)
    def _():
        pltpu.sync_copy(acc.at[pl.ds(row0, rpt)], y0_h.at[pl.ds(row0, rpt)])

    @pl.when(cid == 1)
    def _():
        pltpu.sync_copy(acc.at[pl.ds(row0, rpt)], y1_h.at[pl.ds(row0, rpt)])


def _bd8(w):
    c, d = w.shape
    return jnp.einsum('jk,cd->jckd', jnp.eye(UP, dtype=w.dtype), w).reshape(
        UP * c, UP * d)


def _grouped(a):
    return a.reshape(N_IN, G_HALF)


def _flat16(a):
    return a.reshape(N_UP, HALF)


def kernel(x, edge_index_up, target_label, W_up, b_up, W1_self, W1_nbr, b1,
           Wb_self, Wb_nbr, bb, Wc_self, Wc_nbr, bc):
    f32 = jnp.float32
    src = edge_index_up[0]
    dst = edge_index_up[1]
    zer = jnp.zeros((N_UP, HALF), f32)

    plo = _bd8(jnp.concatenate(
        [jnp.eye(HALF, dtype=f32), jnp.zeros((HALF, HALF), f32)], axis=1))
    phi = _bd8(jnp.concatenate(
        [jnp.zeros((HALF, HALF), f32), jnp.eye(HALF, dtype=f32)], axis=1))

    wf_up = jnp.transpose(W_up, (1, 0, 2)).reshape(C_IN, G_HID)
    bf_up = jnp.tile(b_up, UP).reshape(1, G_HID)
    h = _up(x, wf_up, bf_up)

    s, tlo, thi = _proj0(h, _bd8(W1_self), _bd8(W1_nbr[:, :HALF]),
                         _bd8(W1_nbr[:, HALF:]),
                         jnp.tile(b1, UP).reshape(1, G_OUT))
    ylo, yhi = _sc_wide(_flat16(tlo), _flat16(thi), src, dst, zer)

    hres, s, tlo, thi = _cproj(
        s, _grouped(ylo), _grouped(yhi), plo, phi,
        _bd8(Wb_self[0, 0]), _bd8(Wb_nbr[0, 0][:, :HALF]),
        _bd8(Wb_nbr[0, 0][:, HALF:]), jnp.tile(bb[0, 0], UP).reshape(1, G_OUT))
    ylo, yhi = _sc_wide(_flat16(tlo), _flat16(thi), src, dst, zer)

    for l in range(L_BLOCK):
        _, s, tlo, thi = _cproj(
            s, _grouped(ylo), _grouped(yhi), plo, phi,
            _bd8(Wb_self[l, 1]), _bd8(Wb_nbr[l, 1][:, :HALF]),
            _bd8(Wb_nbr[l, 1][:, HALF:]),
            jnp.tile(bb[l, 1], UP).reshape(1, G_OUT))
        ylo, yhi = _sc_wide(_flat16(tlo), _flat16(thi), src, dst, zer)
        if l < L_BLOCK - 1:
            hres, s, tlo, thi = _cproj_res(
                hres, s, _grouped(ylo), _grouped(yhi), plo, phi,
                _bd8(Wb_self[l + 1, 0]), _bd8(Wb_nbr[l + 1, 0][:, :HALF]),
                _bd8(Wb_nbr[l + 1, 0][:, HALF:]),
                jnp.tile(bb[l + 1, 0], UP).reshape(1, G_OUT))
            ylo, yhi = _sc_wide(_flat16(tlo), _flat16(thi), src, dst, zer)

    e0 = jnp.zeros((1, HALF), f32).at[0, 0].set(1.0)
    wcs = _bd8(Wc_self @ e0)
    wcn = _bd8(Wc_nbr @ e0)
    bc_g = jnp.tile(bc[0] * e0, (1, UP))
    hfin, sc_g, tpad = _ccls(hres, s, _grouped(ylo), _grouped(yhi), plo, phi,
                             wcs, wcn, bc_g)
    y0, y1 = _sc_cls(_flat16(tpad), src, dst, zer)

    lg_g, mx, top = _finreduce(sc_g, _grouped(y0), _grouped(y1))

    out_cls = lg_g.reshape(N_UP, HALF)[:, :1]
    hN = hfin.reshape(N_UP, C_OUT)
    tgtN = target_label.astype(jnp.int32).reshape(N_UP, 1)
    out_pruned, keep_i = _mask(hN, out_cls, tgtN, mx, top)
    keep = keep_i.reshape(N_UP) != 0
    return out_pruned, out_cls, target_label, keep
